# baseline (device time: 45452 ns/iter reference)
import jax
import jax.numpy as jnp
from jax import lax
from jax.experimental import pallas as pl
from jax.experimental.pallas import tpu as pltpu

N_Y = 4
B = 16
H = 16
D = 64
BS = 16
NP = 128
NK = NP * BS


def kernel(Q, K, V, bt, lens):
    lens2 = lens.reshape(B, 1)

    def body(q_ref, k_ref, v_ref, bt_ref, lens_ref, out_ref):
        my_y = lax.axis_index("y")

        q = q_ref[:, 0, :, :].astype(jnp.bfloat16)
        qh = jnp.transpose(q, (1, 0, 2))
        k = k_ref[...].astype(jnp.bfloat16)
        v = v_ref[...].astype(jnp.bfloat16)
        kt = jnp.reshape(jnp.transpose(k, (2, 0, 1, 3)), (H, NK, D))
        vt = jnp.reshape(jnp.transpose(v, (2, 0, 1, 3)), (H, NK, D))

        red = (jnp.sum(kt.astype(jnp.float32), axis=1)
               + jnp.sum(vt.astype(jnp.float32), axis=1))
        res = qh.astype(jnp.float32) + red[:, None, :]
        out_ref[...] = jnp.transpose(res, (1, 0, 2))[:, None, :, :]

    out_shape = jax.ShapeDtypeStruct((B, 1, H, D), jnp.float32)
    return pl.pallas_call(
        body,
        out_shape=out_shape,
        in_specs=[pl.BlockSpec(memory_space=pltpu.VMEM)] * 5,
        out_specs=pl.BlockSpec(memory_space=pltpu.VMEM),
    )(Q, K, V, bt, lens2)


# device time: 43255 ns/iter; 1.0508x vs baseline; 1.0508x over previous
import jax
import jax.numpy as jnp
from jax import lax
from jax.experimental import pallas as pl
from jax.experimental.pallas import tpu as pltpu

N_Y = 4
B = 16
H = 16
D = 64
BS = 16
NP = 128
NK = NP * BS


def kernel(Q, K, V, bt, lens):
    lens2 = lens.reshape(B, 1)

    def body(q_ref, k_ref, v_ref, bt_ref, lens_ref, out_ref):
        my_y = lax.axis_index("y")

        q = q_ref[:, 0, :, :].astype(jnp.bfloat16)
        qh = jnp.transpose(q, (1, 0, 2))
        k = k_ref[...].astype(jnp.bfloat16)
        v = v_ref[...].astype(jnp.bfloat16)
        kt = jnp.reshape(jnp.transpose(k, (2, 0, 1, 3)), (H, NK, D))
        vt = jnp.reshape(jnp.transpose(v, (2, 0, 1, 3)), (H, NK, D))

        red = (jnp.sum(k.astype(jnp.float32), axis=(0, 1))
               + jnp.sum(v.astype(jnp.float32), axis=(0, 1)))
        res = qh.astype(jnp.float32) + red[:, None, :]
        out_ref[...] = jnp.transpose(res, (1, 0, 2))[:, None, :, :]

    out_shape = jax.ShapeDtypeStruct((B, 1, H, D), jnp.float32)
    return pl.pallas_call(
        body,
        out_shape=out_shape,
        in_specs=[pl.BlockSpec(memory_space=pltpu.VMEM)] * 5,
        out_specs=pl.BlockSpec(memory_space=pltpu.VMEM),
    )(Q, K, V, bt, lens2)


# device time: 32680 ns/iter; 1.3908x vs baseline; 1.3236x over previous
import jax
import jax.numpy as jnp
from jax import lax
from jax.experimental import pallas as pl
from jax.experimental.pallas import tpu as pltpu

N_Y = 4
B = 16
H = 16
D = 64
BS = 16
NP = 128
NK = NP * BS


def kernel(Q, K, V, bt, lens):
    lens2 = lens.reshape(B, 1)

    def body(q_ref, k_ref, v_ref, bt_ref, lens_ref, out_ref):
        my_y = lax.axis_index("y")

        q = q_ref[:, 0, :, :].astype(jnp.bfloat16)
        qh = jnp.transpose(q, (1, 0, 2))

        bt_ = bt_ref[...]
        lens_ = lens_ref[...]
        j_ids = lax.broadcasted_iota(jnp.int32, (B, NP, NP), 1)
        p_ids = lax.broadcasted_iota(jnp.int32, (B, NP, NP), 2)
        match = bt_[:, :, None] == (p_ids + my_y * NP)
        valid = j_ids < lens_[:, :, None]
        w = jnp.sum(jnp.where(match & valid, 1.0, 0.0), axis=1)

        res = qh.astype(jnp.float32) + jnp.sum(w)
        out_ref[...] = jnp.transpose(res, (1, 0, 2))[:, None, :, :]

    out_shape = jax.ShapeDtypeStruct((B, 1, H, D), jnp.float32)
    return pl.pallas_call(
        body,
        out_shape=out_shape,
        in_specs=[
            pl.BlockSpec(memory_space=pltpu.VMEM),
            pl.BlockSpec(memory_space=pl.ANY),
            pl.BlockSpec(memory_space=pl.ANY),
            pl.BlockSpec(memory_space=pltpu.VMEM),
            pl.BlockSpec(memory_space=pltpu.VMEM),
        ],
        out_specs=pl.BlockSpec(memory_space=pltpu.VMEM),
    )(Q, K, V, bt, lens2)


# device time: 32189 ns/iter; 1.4120x vs baseline; 1.0153x over previous
import jax
import jax.numpy as jnp
from jax import lax
from jax.experimental import pallas as pl
from jax.experimental.pallas import tpu as pltpu

N_Y = 4
B = 16
H = 16
D = 64
BS = 16
NP = 128
NK = NP * BS


def kernel(Q, K, V, bt, lens):
    lens2 = lens.reshape(B, 1)

    def body(q_ref, k_ref, v_ref, bt_ref, lens_ref, out_ref):
        my_y = lax.axis_index("y")

        q = q_ref[:, 0, :, :].astype(jnp.bfloat16)
        qh = jnp.transpose(q, (1, 0, 2))

        res = qh.astype(jnp.float32) + jnp.float32(my_y)
        out_ref[...] = jnp.transpose(res, (1, 0, 2))[:, None, :, :]

    out_shape = jax.ShapeDtypeStruct((B, 1, H, D), jnp.float32)
    return pl.pallas_call(
        body,
        out_shape=out_shape,
        in_specs=[
            pl.BlockSpec(memory_space=pltpu.VMEM),
            pl.BlockSpec(memory_space=pl.ANY),
            pl.BlockSpec(memory_space=pl.ANY),
            pl.BlockSpec(memory_space=pltpu.VMEM),
            pl.BlockSpec(memory_space=pltpu.VMEM),
        ],
        out_specs=pl.BlockSpec(memory_space=pltpu.VMEM),
    )(Q, K, V, bt, lens2)


# device time: 31167 ns/iter; 1.4583x vs baseline; 1.0328x over previous
import jax
import jax.numpy as jnp
from jax import lax
from jax.experimental import pallas as pl
from jax.experimental.pallas import tpu as pltpu

N_Y = 4
B = 16
H = 16
D = 64
BS = 16
NP = 128


def kernel(Q, K, V, bt, lens):
    lens2 = lens.reshape(B, 1)
    Kt = jnp.transpose(K, (1, 2, 3, 0))
    Vt = jnp.transpose(V, (1, 2, 3, 0))

    def body(q_ref, k_ref, v_ref, bt_ref, lens_ref, out_ref,
             o_comm, ml_comm, send_sems, recv_sems):
        my_x = lax.axis_index("x")
        my_y = lax.axis_index("y")
        my_z = lax.axis_index("z")

        q = q_ref[:, 0, :, :].astype(jnp.bfloat16)
        qh = jnp.transpose(q, (1, 0, 2))
        qb = jnp.reshape(
            jnp.broadcast_to(qh[:, None], (H, BS, B, D)), (H * BS, B, D))
        k = jnp.reshape(
            jnp.transpose(k_ref[...], (1, 0, 2, 3)).astype(jnp.bfloat16),
            (H * BS, D, NP))
        v = jnp.reshape(
            jnp.transpose(v_ref[...], (1, 0, 2, 3)).astype(jnp.bfloat16),
            (H * BS, D, NP))

        s = jnp.reshape(
            lax.dot_general(
                qb, k,
                dimension_numbers=(((2,), (1,)), ((0,), (0,))),
                preferred_element_type=jnp.float32,
            ),
            (H, BS, B, NP),
        ) * (D ** -0.5)

        bt_ = bt_ref[...]
        lens_ = lens_ref[...]
        j_ids = lax.broadcasted_iota(jnp.int32, (B, NP, NP), 1)
        p_ids = lax.broadcasted_iota(jnp.int32, (B, NP, NP), 2)
        match = bt_[:, :, None] == (p_ids + my_y * NP)
        valid = j_ids < lens_[:, :, None]
        w = jnp.sum(jnp.where(match & valid, 1.0, 0.0), axis=1)

        neg = jnp.float32(-1e30)
        s_m = jnp.where((w > 0.0)[None, None, :, :], s, neg)
        m = jnp.max(s_m, axis=(1, 3))
        e = jnp.exp(s_m - m[:, None, :, None]) * w[None, None]
        l = jnp.sum(e, axis=(1, 3))
        o = jnp.sum(
            jnp.reshape(
                lax.dot_general(
                    jnp.reshape(e.astype(jnp.bfloat16), (H * BS, B, NP)), v,
                    dimension_numbers=(((2,), (2,)), ((0,), (0,))),
                    preferred_element_type=jnp.float32,
                ),
                (H, BS, B, D),
            ),
            axis=1,
        )

        o_comm[my_y] = o
        ml_comm[my_y, 0] = m
        ml_comm[my_y, 1] = l

        barrier = pltpu.get_barrier_semaphore()
        for d in (1, 2, 3):
            peer = (my_y + d) % N_Y
            pl.semaphore_signal(
                barrier, inc=1,
                device_id=(my_x, peer, my_z),
                device_id_type=pl.DeviceIdType.MESH,
            )
        pl.semaphore_wait(barrier, N_Y - 1)

        sends = []
        for d in (1, 2, 3):
            peer = (my_y + d) % N_Y
            for bi, buf in ((0, o_comm), (1, ml_comm)):
                rdma = pltpu.make_async_remote_copy(
                    src_ref=buf.at[my_y],
                    dst_ref=buf.at[my_y],
                    send_sem=send_sems.at[bi, d],
                    recv_sem=recv_sems.at[bi, d],
                    device_id=(my_x, peer, my_z),
                    device_id_type=pl.DeviceIdType.MESH,
                )
                rdma.start()
                sends.append(rdma)

        for d in (1, 2, 3):
            src = (my_y - d) % N_Y
            for bi, buf in ((0, o_comm), (1, ml_comm)):
                recv = pltpu.make_async_remote_copy(
                    src_ref=buf.at[src],
                    dst_ref=buf.at[src],
                    send_sem=send_sems.at[bi, d],
                    recv_sem=recv_sems.at[bi, d],
                    device_id=(my_x, src, my_z),
                    device_id_type=pl.DeviceIdType.MESH,
                )
                recv.wait_recv()

        M = ml_comm[0, 0]
        for j in range(1, N_Y):
            M = jnp.maximum(M, ml_comm[j, 0])
        L = jnp.zeros((H, B), jnp.float32)
        O = jnp.zeros((H, B, D), jnp.float32)
        for j in range(N_Y):
            a = jnp.exp(ml_comm[j, 0] - M)
            L = L + a * ml_comm[j, 1]
            O = O + a[:, :, None] * o_comm[j]
        res = O / L[:, :, None]
        out_ref[...] = jnp.transpose(res, (1, 0, 2))[:, None, :, :]

        for rdma in sends:
            rdma.wait_send()

    out_shape = jax.ShapeDtypeStruct((B, 1, H, D), jnp.float32)
    return pl.pallas_call(
        body,
        out_shape=out_shape,
        in_specs=[pl.BlockSpec(memory_space=pltpu.VMEM)] * 5,
        out_specs=pl.BlockSpec(memory_space=pltpu.VMEM),
        scratch_shapes=[
            pltpu.VMEM((N_Y, H, B, D), jnp.float32),
            pltpu.VMEM((N_Y, 2, H, B), jnp.float32),
            pltpu.SemaphoreType.DMA((2, N_Y)),
            pltpu.SemaphoreType.DMA((2, N_Y)),
        ],
        compiler_params=pltpu.CompilerParams(collective_id=0),
    )(Q, Kt, Vt, bt, lens2)
